# Initial kernel scaffold; baseline (speedup 1.0000x reference)
import jax
import jax.numpy as jnp
from jax.experimental import pallas as pl


def kernel(boxes, scores, labels):
    raise NotImplementedError("write your pallas kernel here")


# speedup vs baseline: 4099.6271x; 4099.6271x over previous
import jax
import jax.numpy as jnp
from jax.experimental import pallas as pl
from jax.experimental.pallas import tpu as pltpu

_N = 20000
_R, _C = 160, 128
_PAD = _R * _C - _N
_SCORE_THRESH = 0.2
_NMS_THRESH = 0.5
_MAX_H = 15
_MAX_O = 15


def _nms_body(x1r, y1r, x2r, y2r, sr, lr, outr):
    x1 = x1r[:]; y1 = y1r[:]; x2 = x2r[:]; y2 = y2r[:]
    s = sr[:]; lab = lr[:]
    valid = s >= _SCORE_THRESH
    coordmax = jnp.maximum(jnp.maximum(x1, y1), jnp.maximum(x2, y2))
    bmax = jnp.max(jnp.where(valid, coordmax, -jnp.inf))
    off = lab * (bmax + 1.0)
    ox1 = x1 + off; oy1 = y1 + off; ox2 = x2 + off; oy2 = y2 + off
    areas = (ox2 - ox1) * (oy2 - oy1)
    flat = (jax.lax.broadcasted_iota(jnp.int32, (_R, _C), 0) * _C
            + jax.lax.broadcasted_iota(jnp.int32, (_R, _C), 1))
    orow = jax.lax.broadcasted_iota(jnp.int32, (32, _C), 0)
    ocol = jax.lax.broadcasted_iota(jnp.int32, (32, _C), 1)
    is_h = lab == 0.0

    def pick(state, human):
        active, acc, r = state
        elig = (active > 0.0) & (is_h if human else jnp.logical_not(is_h))
        ms = jnp.where(elig, s, -1.0)
        m = jnp.max(ms)
        hit = elig & (ms == m)
        pidx = jnp.min(jnp.where(hit, flat, jnp.int32(2**30)))
        sel = ((flat == pidx) & hit).astype(jnp.float32)
        prx1 = jnp.sum(x1 * sel); pry1 = jnp.sum(y1 * sel)
        prx2 = jnp.sum(x2 * sel); pry2 = jnp.sum(y2 * sel)
        ps = jnp.sum(s * sel)
        pox1 = jnp.sum(ox1 * sel); poy1 = jnp.sum(oy1 * sel)
        pox2 = jnp.sum(ox2 * sel); poy2 = jnp.sum(oy2 * sel)
        parea = jnp.sum(areas * sel)
        xx1 = jnp.maximum(pox1, ox1); yy1 = jnp.maximum(poy1, oy1)
        xx2 = jnp.minimum(pox2, ox2); yy2 = jnp.minimum(poy2, oy2)
        w = jnp.maximum(0.0, xx2 - xx1); h = jnp.maximum(0.0, yy2 - yy1)
        inter = w * h
        iou = inter / (parea + areas - inter + 1e-9)
        ok = m >= 0.0
        active = jnp.where(ok & (iou > _NMS_THRESH), 0.0, active)
        val = jnp.where(ocol == 0, prx1,
              jnp.where(ocol == 1, pry1,
              jnp.where(ocol == 2, prx2,
              jnp.where(ocol == 3, pry2,
              jnp.where(ocol == 4, ps, 0.0)))))
        acc = jnp.where((orow == r) & ok, val, acc)
        r = r + ok.astype(jnp.int32)
        return active, acc, r

    state = (valid.astype(jnp.float32), jnp.zeros((32, _C), jnp.float32),
             jnp.int32(0))
    state = jax.lax.fori_loop(0, _MAX_H, lambda i, st: pick(st, True), state)
    state = jax.lax.fori_loop(0, _MAX_O, lambda i, st: pick(st, False), state)
    outr[:] = state[1]


def kernel(boxes, scores, labels):
    x1 = jnp.pad(boxes[:, 0], (0, _PAD)).reshape(_R, _C)
    y1 = jnp.pad(boxes[:, 1], (0, _PAD)).reshape(_R, _C)
    x2 = jnp.pad(boxes[:, 2], (0, _PAD)).reshape(_R, _C)
    y2 = jnp.pad(boxes[:, 3], (0, _PAD)).reshape(_R, _C)
    s = jnp.pad(scores, (0, _PAD), constant_values=-1.0).reshape(_R, _C)
    labf = jnp.pad(labels.astype(jnp.float32), (0, _PAD),
                   constant_values=-1.0).reshape(_R, _C)
    res = pl.pallas_call(
        _nms_body,
        out_shape=jax.ShapeDtypeStruct((32, _C), jnp.float32),
    )(x1, y1, x2, y2, s, labf)
    return res[:30, :5]
